# baseline (speedup 1.0000x reference)
import functools

import jax
import jax.numpy as jnp
from jax import lax
from jax.experimental import pallas as pl
from jax.experimental.pallas import tpu as pltpu
from jax.experimental.pallas import tpu_sc as plsc

N_DIM = 16
L = 16
CHUNK = 128


def _rsqrt(x):
    xi = plsc.bitcast(x, jnp.int32)
    y = plsc.bitcast(jnp.int32(0x5F3759DF) - (xi >> 1), jnp.float32)
    for _ in range(3):
        y = y * (1.5 - 0.5 * x * y * y)
    return y


def _make_kernel(batch):
    info = plsc.get_sparse_core_info()
    nc, ns = info.num_cores, info.num_subcores
    nw = nc * ns
    assert batch % (nw * L) == 0
    bpw = batch // nw
    n_chunks = bpw // CHUNK
    n_groups = bpw // L

    mesh = plsc.VectorSubcoreMesh(core_axis_name="c", subcore_axis_name="s")

    @functools.partial(
        pl.kernel,
        mesh=mesh,
        compiler_params=pltpu.CompilerParams(
            needs_layout_passes=False, use_tc_tiling_on_sc=False),
        out_type=jax.ShapeDtypeStruct((batch,), jnp.float32),
        scratch_types=[
            pltpu.VMEM((n_chunks, CHUNK), jnp.int32),
            pltpu.VMEM((n_chunks, CHUNK), jnp.int32),
            pltpu.VMEM((bpw, N_DIM), jnp.float32),
            pltpu.VMEM((bpw, N_DIM), jnp.float32),
            pltpu.VMEM((bpw,), jnp.float32),
            pltpu.VMEM((bpw,), jnp.float32),
            pltpu.VMEM((L,), jnp.float32),
            pltpu.VMEM((bpw,), jnp.float32),
            pltpu.SemaphoreType.DMA,
        ],
    )
    def k(u_hbm, i_hbm, ub_hbm, uv_hbm, ib_hbm, iv_hbm, gb_hbm, out_hbm,
          u_v, i_v, uv_v, iv_v, ub_v, ib_v, gb_v, out_v, sem):
        wid = lax.axis_index("s") * nc + lax.axis_index("c")
        base = wid * bpw

        idx_cps = []
        for c in range(n_chunks):
            idx_cps.append(pltpu.async_copy(
                u_hbm.at[pl.ds(base + c * CHUNK, CHUNK)], u_v.at[c], sem))
            idx_cps.append(pltpu.async_copy(
                i_hbm.at[pl.ds(base + c * CHUNK, CHUNK)], i_v.at[c], sem))
        pltpu.sync_copy(gb_hbm, gb_v.at[pl.ds(0, 1)])
        for cp in idx_cps:
            cp.wait()

        cps = []
        for c in range(n_chunks):
            sl = pl.ds(c * CHUNK, CHUNK)
            cps.append(pltpu.async_copy(uv_hbm.at[u_v.at[c]], uv_v.at[sl], sem))
            cps.append(pltpu.async_copy(iv_hbm.at[i_v.at[c]], iv_v.at[sl], sem))
            cps.append(pltpu.async_copy(ub_hbm.at[u_v.at[c]], ub_v.at[sl], sem))
            cps.append(pltpu.async_copy(ib_hbm.at[i_v.at[c]], ib_v.at[sl], sem))
        for cp in cps:
            cp.wait()

        iota = lax.iota(jnp.int32, L)
        gb = gb_v[...][0]

        def group(g, carry):
            rows = g * L + iota
            bu = ub_v[pl.ds(g * L, L)]
            bi = ib_v[pl.ds(g * L, L)]
            s_u = jnp.zeros((L,), jnp.float32)
            s_i = jnp.zeros((L,), jnp.float32)
            t = jnp.zeros((L,), jnp.float32)
            for d in range(N_DIM):
                col = jnp.full((L,), d, jnp.int32)
                lu = plsc.load_gather(uv_v, [rows, col])
                li = plsc.load_gather(iv_v, [rows, col])
                eu = jnp.exp(0.5 * lu)
                ei = jnp.exp(0.5 * li)
                s_u = s_u + eu * eu
                s_i = s_i + ei * ei
                t = t + eu * ei
            bc = t * _rsqrt(s_u * s_i)
            z = jnp.maximum(1.0 - bc, 1e-36)
            intx = z * _rsqrt(z)
            out_v[pl.ds(g * L, L)] = bu + bi + intx + gb
            return carry

        lax.fori_loop(0, n_groups, group, 0)
        pltpu.sync_copy(out_v, out_hbm.at[pl.ds(base, bpw)])

    return k


def kernel(u, i, user_bias, user_vect, item_bias, item_vect, glob_bias):
    batch = u.shape[0]
    k = _make_kernel(batch)
    return k(u.astype(jnp.int32), i.astype(jnp.int32),
             user_bias.reshape(-1), user_vect, item_bias.reshape(-1),
             item_vect, glob_bias)
